# baseline (speedup 1.0000x reference)
import numpy as np

import jax
import jax.numpy as jnp
from jax.experimental import pallas as pl

_E_TILE = 256


def _row_prod(x):
    n = x.shape[-1]
    while n > 128:
        half = n // 2
        x = x[:, :half] * x[:, half:n]
        n = half
    while n > 1:
        half = n // 2
        x = x[:, :half] * x[:, half:n]
        n = half
    return x


def _bp_iter_kernel(c_ref, mu_ref, tot_ref, hbase_ref, s2_ref,
                    mu_out_ref, tot_out_ref):
    j = pl.program_id(0)
    c = c_ref[...]
    mask = c >= 0.0
    wv = jnp.maximum(c, 0.0)
    contrib = mu_ref[...] * wv
    m = (hbase_ref[...] + tot_ref[...]) - contrib
    t = jnp.where(mask, jnp.tanh(m), 1.0)
    is_zero = t == 0.0
    nzv = jnp.where(is_zero, 1.0, t)
    prod_nz = _row_prod(nzv)
    zero_cnt = jnp.sum(is_zero.astype(jnp.float32), axis=1, keepdims=True)
    iz_f = jnp.where(is_zero, 1.0, 0.0)
    sel = (zero_cnt - iz_f) == 0.0
    ratio = (nzv + prod_nz) / (nzv - prod_nz)
    mu_new = jnp.where(sel, s2_ref[...] * jnp.log2(ratio), 0.0)
    mu_out_ref[...] = mu_new
    part = jnp.sum(mu_new * wv, axis=0, keepdims=True)

    @pl.when(j == 0)
    def _():
        tot_out_ref[...] = jnp.zeros_like(tot_out_ref)

    tot_out_ref[...] += part


def _bp_iteration(c, base2d, sign2d, mu, tot):
    num_edges, num_nodes = c.shape
    t = _E_TILE
    grid = (num_edges // t,)
    return pl.pallas_call(
        _bp_iter_kernel,
        grid=grid,
        in_specs=[
            pl.BlockSpec((t, num_nodes), lambda j: (j, 0)),
            pl.BlockSpec((t, num_nodes), lambda j: (j, 0)),
            pl.BlockSpec((1, num_nodes), lambda j: (0, 0)),
            pl.BlockSpec((1, num_nodes), lambda j: (0, 0)),
            pl.BlockSpec((t, 1), lambda j: (j, 0)),
        ],
        out_specs=[
            pl.BlockSpec((t, num_nodes), lambda j: (j, 0)),
            pl.BlockSpec((1, num_nodes), lambda j: (0, 0)),
        ],
        out_shape=[
            jax.ShapeDtypeStruct((num_edges, num_nodes), jnp.float32),
            jax.ShapeDtypeStruct((1, num_nodes), jnp.float32),
        ],
    )(c, mu, tot, base2d, sign2d)


def kernel(l_v, h, s_c, iterations, b, w):
    num_edges, num_nodes = h.shape
    hbase2d = (0.5 * l_v * b).reshape(1, num_nodes)
    s2 = ((1.0 - 2.0 * s_c.astype(jnp.float32))
          * (0.5 * float(np.log(2.0)))).reshape(num_edges, 1)
    c = jnp.where(h == 1, w, -1.0).astype(jnp.float32)

    mu0 = jnp.zeros((num_edges, num_nodes), jnp.float32)
    tot0 = jnp.zeros((1, num_nodes), jnp.float32)

    def body(_, state):
        mu, tot = state
        mu_new, tot_new = _bp_iteration(c, hbase2d, s2, mu, tot)
        return (mu_new, tot_new)

    _, tot = jax.lax.fori_loop(0, iterations, body, (mu0, tot0))
    mu_v = 2.0 * (hbase2d[0] + tot[0])
    return 1.0 / (jnp.exp(mu_v) + 1.0)


# speedup vs baseline: 1.6446x; 1.6446x over previous
import numpy as np

import jax
import jax.numpy as jnp
from jax.experimental import pallas as pl
from jax.experimental.pallas import tpu as pltpu

_E_TILE = 256


def _row_prod(x):
    n = x.shape[-1]
    while n > 1:
        half = n // 2
        x = x[:, :half] * x[:, half:n]
        n = half
    return x


def _bp_kernel(c_ref, hbase_ref, s2_ref, out_ref, mu_s, tot_s):
    i = pl.program_id(0)
    j = pl.program_id(1)
    n_iter = pl.num_programs(0)
    n_tiles = pl.num_programs(1)
    t = _E_TILE
    rows = pl.ds(j * t, t)

    @pl.when(jnp.logical_and(i == 0, j == 0))
    def _():
        tot_s[0] = jnp.zeros_like(tot_s[0])

    @pl.when(j == 0)
    def _():
        tot_s[(i + 1) % 2] = jnp.zeros_like(tot_s[0])

    @pl.when(i == 0)
    def _():
        mu_s[rows, :] = jnp.zeros((t, mu_s.shape[1]), jnp.float32)

    c = c_ref[...]
    mask = c >= 0.0
    wv = jnp.maximum(c, 0.0)
    contrib = mu_s[rows, :] * wv
    m = (hbase_ref[...] + tot_s[i % 2]) - contrib
    tt = jnp.where(mask, jnp.tanh(m), 1.0)
    is_zero = tt == 0.0
    nzv = jnp.where(is_zero, 1.0, tt)
    prod_nz = _row_prod(nzv)
    zero_cnt = jnp.sum(is_zero.astype(jnp.float32), axis=1, keepdims=True)
    iz_f = jnp.where(is_zero, 1.0, 0.0)
    sel = (zero_cnt - iz_f) == 0.0
    ratio = (nzv + prod_nz) / (nzv - prod_nz)
    mu_new = jnp.where(sel, s2_ref[...] * jnp.log2(ratio), 0.0)
    mu_s[rows, :] = mu_new
    part = jnp.sum(mu_new * wv, axis=0, keepdims=True)
    tot_s[(i + 1) % 2] += part

    @pl.when(jnp.logical_and(i == n_iter - 1, j == n_tiles - 1))
    def _():
        out_ref[...] = tot_s[n_iter % 2]


def kernel(l_v, h, s_c, iterations, b, w):
    num_edges, num_nodes = h.shape
    hbase2d = (0.5 * l_v * b).reshape(1, num_nodes)
    s2 = ((1.0 - 2.0 * s_c.astype(jnp.float32))
          * (0.5 * float(np.log(2.0)))).reshape(num_edges, 1)
    c = jnp.where(h == 1, w, -1.0).astype(jnp.float32)

    t = _E_TILE
    n_tiles = num_edges // t
    tot = pl.pallas_call(
        _bp_kernel,
        grid=(iterations, n_tiles),
        in_specs=[
            pl.BlockSpec((t, num_nodes), lambda i, j: (j, 0)),
            pl.BlockSpec((1, num_nodes), lambda i, j: (0, 0)),
            pl.BlockSpec((t, 1), lambda i, j: (j, 0)),
        ],
        out_specs=pl.BlockSpec((1, num_nodes), lambda i, j: (0, 0)),
        out_shape=jax.ShapeDtypeStruct((1, num_nodes), jnp.float32),
        scratch_shapes=[
            pltpu.VMEM((num_edges, num_nodes), jnp.float32),
            pltpu.VMEM((2, 1, num_nodes), jnp.float32),
        ],
    )(c, hbase2d, s2)

    mu_v = 2.0 * (hbase2d[0] + tot[0])
    return 1.0 / (jnp.exp(mu_v) + 1.0)


# speedup vs baseline: 1.7367x; 1.0560x over previous
import numpy as np

import jax
import jax.numpy as jnp
from jax.experimental import pallas as pl
from jax.experimental.pallas import tpu as pltpu

_E_TILE = 256
_ROWS = 8


def _row_reduce(x, op):
    n = x.shape[-1]
    while n > 1:
        half = n // 2
        x = op(x[:, :half], x[:, half:n])
        n = half
    return x


def _bp_kernel(c_ref, hbase_ref, s2_ref, out_ref, mu_s, tot_s, bt_s, stage_s):
    i = pl.program_id(0)
    j = pl.program_id(1)
    n_iter = pl.num_programs(0)
    n_tiles = pl.num_programs(1)
    t = _E_TILE
    r_sz = _ROWS
    num_nodes = c_ref.shape[1]

    @pl.when(jnp.logical_and(i == 0, j == 0))
    def _():
        tot_s[0] = jnp.zeros_like(tot_s[0])

    @pl.when(j == 0)
    def _():
        tot_s[(i + 1) % 2] = jnp.zeros_like(tot_s[0])

    @pl.when(i == 0)
    def _():
        mu_s[pl.ds(j * t, t), :] = jnp.zeros((t, num_nodes), jnp.float32)

    bt_s[...] = hbase_ref[...] + tot_s[i % 2]
    pacc = jnp.zeros((r_sz, num_nodes), jnp.float32)

    for r in range(t // r_sz):
        rows_in = pl.ds(r * r_sz, r_sz)
        rows_mu = pl.ds(j * t + r * r_sz, r_sz)
        c8 = c_ref[rows_in, :]
        wv = jnp.maximum(c8, 0.0)
        mn = jnp.minimum(c8, 0.0)
        m = (bt_s[...] - mu_s[rows_mu, :] * wv) - mn
        tt = jnp.tanh(m)
        izf = jnp.where(tt == 0.0, 1.0, 0.0)
        nzv = tt + izf
        p8 = _row_reduce(nzv, jnp.multiply)
        zc8 = _row_reduce(izf, jnp.add)
        selm = (zc8 - izf) == 0.0
        ratio = (nzv + p8) / (nzv - p8)
        mu_new = jnp.where(selm, s2_ref[rows_in, :] * jnp.log2(ratio), 0.0)
        stage_s[rows_in, :] = mu_new
        pacc = pacc + mu_new * wv

    mu_s[pl.ds(j * t, t), :] = stage_s[...]
    tot_s[(i + 1) % 2] += jnp.sum(pacc, axis=0, keepdims=True)

    @pl.when(jnp.logical_and(i == n_iter - 1, j == n_tiles - 1))
    def _():
        out_ref[...] = tot_s[n_iter % 2]


def kernel(l_v, h, s_c, iterations, b, w):
    num_edges, num_nodes = h.shape
    hbase2d = (0.5 * l_v * b).reshape(1, num_nodes)
    s2 = ((1.0 - 2.0 * s_c.astype(jnp.float32))
          * (0.5 * float(np.log(2.0)))).reshape(num_edges, 1)
    c = jnp.where(h == 1, w, -1e30).astype(jnp.float32)

    t = _E_TILE
    n_tiles = num_edges // t
    tot = pl.pallas_call(
        _bp_kernel,
        grid=(iterations, n_tiles),
        in_specs=[
            pl.BlockSpec((t, num_nodes), lambda i, j: (j, 0)),
            pl.BlockSpec((1, num_nodes), lambda i, j: (0, 0)),
            pl.BlockSpec((t, 1), lambda i, j: (j, 0)),
        ],
        out_specs=pl.BlockSpec((1, num_nodes), lambda i, j: (0, 0)),
        out_shape=jax.ShapeDtypeStruct((1, num_nodes), jnp.float32),
        scratch_shapes=[
            pltpu.VMEM((num_edges, num_nodes), jnp.float32),
            pltpu.VMEM((2, 1, num_nodes), jnp.float32),
            pltpu.VMEM((1, num_nodes), jnp.float32),
            pltpu.VMEM((_E_TILE, num_nodes), jnp.float32),
        ],
    )(c, hbase2d, s2)

    mu_v = 2.0 * (hbase2d[0] + tot[0])
    return 1.0 / (jnp.exp(mu_v) + 1.0)
